# Initial kernel scaffold; baseline (speedup 1.0000x reference)
import jax
import jax.numpy as jnp
from jax.experimental import pallas as pl


def kernel(x_s, edge_index_s, x_p, edge_index_p, x_s_batch, x_p_batch, dose, time, drop_pert, W_c1, b_c1, W_c2, b_c2, W_g1, b_g1, W_g2, b_g2, W_lx, b_lx, W_le, b_le, W_ld, b_ld, W_lt, b_lt, W_l1, b_l1, W_l2, b_l2):
    raise NotImplementedError("write your pallas kernel here")


# speedup vs baseline: 1.7648x; 1.7648x over previous
import functools

import jax
import jax.numpy as jnp
from jax import lax
from jax.experimental import pallas as pl
from jax.experimental.pallas import tpu as pltpu
from jax.experimental.pallas import tpu_sc as plsc

def _elu(x):
    return jnp.where(x > 0, x, jnp.exp(jnp.minimum(x, 0.0)) - 1.0)


N_PAD = 51200
RPT = N_PAD // 16
BM = 512
CHUNK = 96
CPT = 521
E_PAD = 16 * CPT * CHUNK
NB = 64
NRANGE = 16
RNG = N_PAD // NRANGE
RPR = RNG // 16
WCH = 40
NBUF = 2


def _store_groups(o_ref, h, ng):
    for g in range(ng):
        o_ref[g] = h[:, g * 128:(g + 1) * 128]


def _load_groups(a_ref, ng):
    if ng == 1:
        return a_ref[0]
    return jnp.concatenate([a_ref[g] for g in range(ng)], axis=1)


def _tc_first(x, W, deg):
    F = W.shape[1]
    NG = F // 128

    def body(x_ref, w_ref, deg_ref, o_ref):
        dinv = lax.rsqrt(deg_ref[...] + 1.0)
        h = jnp.dot(x_ref[...], w_ref[...], preferred_element_type=jnp.float32)
        _store_groups(o_ref, h * dinv, NG)

    return pl.pallas_call(
        body,
        grid=(N_PAD // BM,),
        in_specs=[
            pl.BlockSpec((BM, x.shape[1]), lambda i: (i, 0)),
            pl.BlockSpec(W.shape, lambda i: (0, 0)),
            pl.BlockSpec((BM, 1), lambda i: (i, 0)),
        ],
        out_specs=pl.BlockSpec((NG, BM, 128), lambda i: (0, i, 0)),
        out_shape=jax.ShapeDtypeStruct((NG, N_PAD, 128), jnp.float32),
    )(x, W, deg)


def _tc_mid(A, W, b, deg):
    FI = W.shape[0]
    F = W.shape[1]
    NGI = FI // 128
    NG = F // 128

    def body(a_ref, w_ref, b_ref, deg_ref, o_ref):
        dinv = lax.rsqrt(deg_ref[...] + 1.0)
        act = _elu(_load_groups(a_ref, NGI) * dinv + b_ref[...])
        h = jnp.dot(act, w_ref[...], preferred_element_type=jnp.float32)
        _store_groups(o_ref, h * dinv, NG)

    return pl.pallas_call(
        body,
        grid=(N_PAD // BM,),
        in_specs=[
            pl.BlockSpec((NGI, BM, 128), lambda i: (0, i, 0)),
            pl.BlockSpec(W.shape, lambda i: (0, 0)),
            pl.BlockSpec((1, F), lambda i: (0, 0)),
            pl.BlockSpec((BM, 1), lambda i: (i, 0)),
        ],
        out_specs=pl.BlockSpec((NG, BM, 128), lambda i: (0, i, 0)),
        out_shape=jax.ShapeDtypeStruct((NG, N_PAD, 128), jnp.float32),
    )(A, W, b, deg)


def _tc_pool(A, b, deg, batch):
    NGI = A.shape[0]
    F = NGI * 128

    def body(a_ref, b_ref, deg_ref, bat_ref, sums_ref, cnt_ref):
        i = pl.program_id(0)

        @pl.when(i == 0)
        def _():
            sums_ref[...] = jnp.zeros_like(sums_ref)
            cnt_ref[...] = jnp.zeros_like(cnt_ref)

        dinv = lax.rsqrt(deg_ref[...] + 1.0)
        act = _elu(_load_groups(a_ref, NGI) * dinv + b_ref[...])
        ids = lax.broadcasted_iota(jnp.int32, (BM, NB), 1)
        oh = (bat_ref[...] == ids).astype(jnp.float32)
        dn = (((0,), (0,)), ((), ()))
        sums_ref[...] += lax.dot_general(oh, act, dn,
                                         preferred_element_type=jnp.float32)
        cnt_ref[...] += lax.dot_general(oh, jnp.ones((BM, 1), jnp.float32), dn,
                                        preferred_element_type=jnp.float32)

    return pl.pallas_call(
        body,
        grid=(N_PAD // BM,),
        in_specs=[
            pl.BlockSpec((NGI, BM, 128), lambda i: (0, i, 0)),
            pl.BlockSpec((1, F), lambda i: (0, 0)),
            pl.BlockSpec((BM, 1), lambda i: (i, 0)),
            pl.BlockSpec((BM, 1), lambda i: (i, 0)),
        ],
        out_specs=[
            pl.BlockSpec((NB, F), lambda i: (0, 0)),
            pl.BlockSpec((NB, 1), lambda i: (0, 0)),
        ],
        out_shape=[
            jax.ShapeDtypeStruct((NB, F), jnp.float32),
            jax.ShapeDtypeStruct((NB, 1), jnp.float32),
        ],
    )(A, b, deg, batch)


def _tc_head(sums_s, cnt_s, sums_p, cnt_p, dose, time,
             W_lx, b_lx, W_le, b_le, W_ld, b_ld, W_lt, b_lt,
             W1x, W1e, w1d, w1t, b_l1, W_l2p, b_l2p):
    OUTP = W_l2p.shape[1]

    def body(ss_ref, cs_ref, sp_ref, cp_ref, do_ref, ti_ref,
             wlx_ref, blx_ref, wle_ref, ble_ref, wld_ref, bld_ref,
             wlt_ref, blt_ref, w1x_ref, w1e_ref, w1d_ref, w1t_ref,
             bl1_ref, wl2_ref, bl2_ref, o_ref):
        xs = ss_ref[...] / jnp.maximum(cs_ref[...], 1.0)
        xp = sp_ref[...] / jnp.maximum(cp_ref[...], 1.0)
        x = jnp.dot(xs, wlx_ref[...], preferred_element_type=jnp.float32) \
            + blx_ref[...]
        e = jnp.dot(xp, wle_ref[...], preferred_element_type=jnp.float32) \
            + ble_ref[...]
        d = do_ref[...] * wld_ref[...] + bld_ref[...]
        t = ti_ref[...] * wlt_ref[...] + blt_ref[...]
        h = (jnp.dot(x, w1x_ref[...], preferred_element_type=jnp.float32)
             + jnp.dot(e, w1e_ref[...], preferred_element_type=jnp.float32)
             + jnp.dot(d, w1d_ref[...], preferred_element_type=jnp.float32)
             + jnp.dot(t, w1t_ref[...], preferred_element_type=jnp.float32)
             + bl1_ref[...])
        h = _elu(h)
        o_ref[...] = jnp.dot(h, wl2_ref[...],
                             preferred_element_type=jnp.float32) + bl2_ref[...]

    return pl.pallas_call(
        body,
        out_shape=jax.ShapeDtypeStruct((NB, OUTP), jnp.float32),
    )(sums_s, cnt_s, sums_p, cnt_p, dose, time,
      W_lx, b_lx, W_le, b_le, W_ld, b_ld, W_lt, b_lt,
      W1x, W1e, w1d, w1t, b_l1, W_l2p, b_l2p)


_MESH = dict(core_axis_name="c", subcore_axis_name="s")


def _sc_degree(dst_s2, dst_p2):
    mesh = plsc.VectorSubcoreMesh(**_MESH)

    @functools.partial(
        pl.kernel,
        out_type=(jax.ShapeDtypeStruct((N_PAD,), jnp.float32),
                  jax.ShapeDtypeStruct((N_PAD,), jnp.float32)),
        mesh=mesh,
        scratch_types=[
            pltpu.VMEM((CPT, CHUNK), jnp.int32),
            pltpu.VMEM((RPT,), jnp.float32),
            pltpu.VMEM((CHUNK,), jnp.float32),
            pltpu.VMEM_SHARED((N_PAD,), jnp.float32),
            pltpu.SemaphoreType.DMA,
        ],
    )
    def k(ds_ref, dp_ref, degs_ref, degp_ref, idx_v, zbuf, obuf, acc, sem):
        c = lax.axis_index("c")
        t = lax.axis_index("s")

        def fz(i, _):
            zbuf[pl.ds(i * 16, 16)] = jnp.zeros((16,), jnp.float32)
            return 0

        lax.fori_loop(0, RPT // 16, fz, 0)
        for i in range(CHUNK // 16):
            obuf[pl.ds(i * 16, 16)] = jnp.ones((16,), jnp.float32)
        pltpu.sync_copy(zbuf, acc.at[pl.ds(t * RPT, RPT)])

        for core in range(2):
            @pl.when(c == core)
            def _(core=core):
                din = ds_ref if core == 0 else dp_ref
                dout = degs_ref if core == 0 else degp_ref
                pltpu.sync_copy(din.at[t], idx_v)
                plsc.subcore_barrier()

                def fs(o, _):
                    descs = []
                    for b in range(8):
                        dst = acc.at[plsc.Indices(idx_v.at[o * 8 + b],
                                                  ignored_value=-1)]
                        descs.append(pltpu.async_copy(obuf, dst, sem,
                                                      add=True))
                    for d in descs:
                        d.wait()
                    return 0

                lax.fori_loop(0, CPT // 8, fs, 0)
                for i in range((CPT // 8) * 8, CPT):
                    dst = acc.at[plsc.Indices(idx_v.at[i], ignored_value=-1)]
                    pltpu.async_copy(obuf, dst, sem, add=True).wait()
                plsc.subcore_barrier()
                pltpu.sync_copy(acc.at[pl.ds(t * RPT, RPT)], zbuf)
                pltpu.sync_copy(zbuf, dout.at[pl.ds(t * RPT, RPT)])

    return k(dst_s2, dst_p2)


def _two_hop(n_chunks, nbuf, a_start, a_wait, b_start, b_wait):
    q = n_chunks // nbuf
    r = n_chunks - q * nbuf
    if q > 0:
        for b in range(nbuf):
            a_start(b, b)
        if q > 1:
            def step(j, _):
                base = j * nbuf
                for b in range(nbuf):
                    b_wait(base - nbuf + b, b)
                    a_start(base + b, b)
                for b in range(nbuf):
                    a_wait(base + b, b)
                    b_start(base + b, b)
                return 0

            for b in range(nbuf):
                a_wait(b, b)
                b_start(b, b)
            lax.fori_loop(1, q, step, 0)
        else:
            for b in range(nbuf):
                a_wait(b, b)
                b_start(b, b)
        for b in range(nbuf):
            b_wait((q - 1) * nbuf + b, b)
    for i in range(r):
        c = q * nbuf + i
        a_start(c, i)
        a_wait(c, i)
        b_start(c, i)
        b_wait(c, i)


def _sc_agg(hn, packed):
    NG = hn.shape[0]
    mesh = plsc.VectorSubcoreMesh(**_MESH)

    @functools.partial(
        pl.kernel,
        out_type=jax.ShapeDtypeStruct((NG, N_PAD, 128), jnp.float32),
        mesh=mesh,
        scratch_types=[
            pltpu.VMEM((CPT, CHUNK), jnp.int32),
            pltpu.VMEM((NBUF, CHUNK), jnp.int32),
            pltpu.VMEM((NBUF, CHUNK), jnp.int32),
            pltpu.VMEM((NBUF, CHUNK, 128), jnp.float32),
            pltpu.VMEM_SHARED((RNG, 128), jnp.float32),
        ] + [pltpu.SemaphoreType.DMA] * (2 * NBUF),
    )
    def k(hn_ref, pk_ref, out_ref, pkv, sbuf, dbuf, rows, acc, *sems):
        gsem = sems[:NBUF]
        ssem = sems[NBUF:]
        c = lax.axis_index("c")
        t = lax.axis_index("s")

        for ri in range(NRANGE // 2):
            r = c * (NRANGE // 2) + ri
            pltpu.sync_copy(pk_ref.at[r, t], pkv)

            for g in range(NG):
                def rrange(i):
                    return pl.ds(r * RNG + t * RPR + i * WCH, WCH)

                def arange_(i):
                    return pl.ds(t * RPR + i * WCH, WCH)

                def wrows(b):
                    return rows.at[b].at[pl.ds(0, WCH)]

                def ia(i, b, wait=False):
                    d = pltpu.make_async_copy(hn_ref.at[g, rrange(i)],
                                              wrows(b), gsem[b])
                    d.wait() if wait else d.start()

                def ib(i, b, wait=False):
                    d = pltpu.make_async_copy(wrows(b), acc.at[arange_(i)],
                                              ssem[b])
                    d.wait() if wait else d.start()

                _two_hop(RPR // WCH, NBUF,
                         lambda i, b: ia(i, b), lambda i, b: ia(i, b, True),
                         lambda i, b: ib(i, b), lambda i, b: ib(i, b, True))
                plsc.subcore_barrier()

                def grows(b):
                    return rows.at[b]

                def gather_start(i, b):
                    for kk in range(CHUNK // 16):
                        sl = pl.ds(kk * 16, 16)
                        w = pkv[i, sl]
                        sbuf[b, sl] = lax.bitwise_and(w, 0xFFFF)
                        dbuf[b, sl] = lax.shift_right_arithmetic(w, 16)
                    idx = plsc.Indices(sbuf.at[b], ignored_value=0xFFFF)
                    pltpu.async_copy(hn_ref.at[g].at[idx], grows(b), gsem[b])

                def gather_wait(i, b):
                    idx = plsc.Indices(sbuf.at[b], ignored_value=0xFFFF)
                    pltpu.make_async_copy(hn_ref.at[g].at[idx], grows(b),
                                          gsem[b]).wait()

                def scatter_start(i, b):
                    dst = acc.at[plsc.Indices(dbuf.at[b], ignored_value=-1)]
                    pltpu.async_copy(grows(b), dst, ssem[b], add=True)

                def scatter_wait(i, b):
                    dst = acc.at[plsc.Indices(dbuf.at[b], ignored_value=-1)]
                    pltpu.make_async_copy(grows(b), dst, ssem[b]).wait()

                _two_hop(CPT, NBUF, gather_start, gather_wait,
                         scatter_start, scatter_wait)
                plsc.subcore_barrier()

                def wa(i, b, wait=False):
                    d = pltpu.make_async_copy(acc.at[arange_(i)], wrows(b),
                                              gsem[b])
                    d.wait() if wait else d.start()

                def wb(i, b, wait=False):
                    d = pltpu.make_async_copy(wrows(b),
                                              out_ref.at[g, rrange(i)],
                                              ssem[b])
                    d.wait() if wait else d.start()

                _two_hop(RPR // WCH, NBUF,
                         lambda i, b: wa(i, b), lambda i, b: wa(i, b, True),
                         lambda i, b: wb(i, b), lambda i, b: wb(i, b, True))
                plsc.subcore_barrier()

    return k(hn, packed)


def _prep_edges(edge_index):
    e = edge_index.shape[1]
    src = jnp.pad(edge_index[0], (0, E_PAD - e), constant_values=0)
    dst = jnp.pad(edge_index[1], (0, E_PAD - e), constant_values=-1)
    rid = jnp.arange(NRANGE, dtype=jnp.int32)[:, None]
    in_rng = (dst[None, :] >= rid * RNG) & (dst[None, :] < (rid + 1) * RNG)
    word = jnp.where(in_rng,
                     ((dst[None, :] - rid * RNG) << 16) | src[None, :],
                     jnp.int32(-1))
    return (dst.reshape(16, CPT, CHUNK),
            word.reshape(NRANGE, 16, CPT, CHUNK))


def _tower(x, packed, deg, W1, b1, W2, b2, batch):
    hn = _tc_first(x, W1, deg)
    A = _sc_agg(hn, packed)
    hn = _tc_mid(A, W2[0], b1.reshape(1, -1), deg)
    A = _sc_agg(hn, packed)
    hn = _tc_mid(A, W2[1], b2[0].reshape(1, -1), deg)
    A = _sc_agg(hn, packed)
    return _tc_pool(A, b2[1].reshape(1, -1), deg, batch)


def kernel(x_s, edge_index_s, x_p, edge_index_p, x_s_batch, x_p_batch,
           dose, time, drop_pert,
           W_c1, b_c1, W_c2, b_c2, W_g1, b_g1, W_g2, b_g2,
           W_lx, b_lx, W_le, b_le, W_ld, b_ld, W_lt, b_lt,
           W_l1, b_l1, W_l2, b_l2):
    n_s = x_s.shape[0]
    n_p = x_p.shape[0]
    xs = jnp.pad(x_s, ((0, N_PAD - n_s), (0, 0)))
    xp = jnp.pad(x_p, ((0, N_PAD - n_p), (0, 6)))
    Wg1 = jnp.pad(W_g1, ((0, 6), (0, 0)))
    bat_s = jnp.pad(x_s_batch, (0, N_PAD - n_s),
                    constant_values=NB).reshape(N_PAD, 1)
    bat_p = jnp.pad(x_p_batch, (0, N_PAD - n_p),
                    constant_values=NB).reshape(N_PAD, 1)
    dst_s2, packed_s = _prep_edges(edge_index_s)
    dst_p2, packed_p = _prep_edges(edge_index_p)

    deg_s, deg_p = _sc_degree(dst_s2, dst_p2)
    deg_s = deg_s.reshape(N_PAD, 1)
    deg_p = deg_p.reshape(N_PAD, 1)

    sums_s, cnt_s = _tower(xs, packed_s, deg_s, W_c1, b_c1, W_c2, b_c2,
                           bat_s)
    sums_p, cnt_p = _tower(xp, packed_p, deg_p, Wg1, b_g1, W_g2, b_g2,
                           bat_p)

    OUT = W_l2.shape[1]
    OUTP = ((OUT + 127) // 128) * 128
    W_l2p = jnp.pad(W_l2, ((0, 0), (0, OUTP - OUT)))
    b_l2p = jnp.pad(b_l2, (0, OUTP - OUT)).reshape(1, OUTP)

    D = W_lx.shape[1]
    out = _tc_head(sums_s, cnt_s, sums_p, cnt_p, dose, time,
                   W_lx, b_lx.reshape(1, -1), W_le, b_le.reshape(1, -1),
                   W_ld, b_ld.reshape(1, 1), W_lt, b_lt.reshape(1, 1),
                   W_l1[:D], W_l1[D:2 * D], W_l1[2 * D:2 * D + 1],
                   W_l1[2 * D + 1:2 * D + 2], b_l1.reshape(1, -1),
                   W_l2p, b_l2p)
    return out[:, :OUT]


# speedup vs baseline: 3.9862x; 2.2588x over previous
import functools

import jax
import jax.numpy as jnp
from jax import lax
from jax.experimental import pallas as pl
from jax.experimental.pallas import tpu as pltpu
from jax.experimental.pallas import tpu_sc as plsc

def _elu(x):
    return jnp.where(x > 0, x, jnp.exp(jnp.minimum(x, 0.0)) - 1.0)


N_PAD = 51200
RPT = N_PAD // 16
BM = 512
CHUNK = 128
CPT = 393
E_PAD = 16 * CPT * CHUNK
NB = 64
NRANGE = 8
RNG = N_PAD // NRANGE
RPR = RNG // 16
WCH = 80
NBUF = 3


def _store_groups(o_ref, h, ng):
    for g in range(ng):
        o_ref[g] = h[:, g * 128:(g + 1) * 128]


def _load_groups(a_ref, ng):
    if ng == 1:
        return a_ref[0]
    return jnp.concatenate([a_ref[g] for g in range(ng)], axis=1)


def _tc_first(x, W, deg):
    F = W.shape[1]
    NG = F // 128

    def body(x_ref, w_ref, deg_ref, o_ref):
        dinv = lax.rsqrt(deg_ref[...] + 1.0)
        h = jnp.dot(x_ref[...], w_ref[...], preferred_element_type=jnp.float32)
        _store_groups(o_ref, h * dinv, NG)

    return pl.pallas_call(
        body,
        grid=(N_PAD // BM,),
        in_specs=[
            pl.BlockSpec((BM, x.shape[1]), lambda i: (i, 0)),
            pl.BlockSpec(W.shape, lambda i: (0, 0)),
            pl.BlockSpec((BM, 1), lambda i: (i, 0)),
        ],
        out_specs=pl.BlockSpec((NG, BM, 128), lambda i: (0, i, 0)),
        out_shape=jax.ShapeDtypeStruct((NG, N_PAD, 128), jnp.float32),
    )(x, W, deg)


def _tc_mid(A, W, b, deg):
    FI = W.shape[0]
    F = W.shape[1]
    NGI = FI // 128
    NG = F // 128

    def body(a_ref, w_ref, b_ref, deg_ref, o_ref):
        dinv = lax.rsqrt(deg_ref[...] + 1.0)
        act = _elu(_load_groups(a_ref, NGI) * dinv + b_ref[...])
        h = jnp.dot(act, w_ref[...], preferred_element_type=jnp.float32)
        _store_groups(o_ref, h * dinv, NG)

    return pl.pallas_call(
        body,
        grid=(N_PAD // BM,),
        in_specs=[
            pl.BlockSpec((NGI, BM, 128), lambda i: (0, i, 0)),
            pl.BlockSpec(W.shape, lambda i: (0, 0)),
            pl.BlockSpec((1, F), lambda i: (0, 0)),
            pl.BlockSpec((BM, 1), lambda i: (i, 0)),
        ],
        out_specs=pl.BlockSpec((NG, BM, 128), lambda i: (0, i, 0)),
        out_shape=jax.ShapeDtypeStruct((NG, N_PAD, 128), jnp.float32),
    )(A, W, b, deg)


def _tc_pool(A, b, deg, batch):
    NGI = A.shape[0]
    F = NGI * 128

    def body(a_ref, b_ref, deg_ref, bat_ref, sums_ref, cnt_ref):
        i = pl.program_id(0)

        @pl.when(i == 0)
        def _():
            sums_ref[...] = jnp.zeros_like(sums_ref)
            cnt_ref[...] = jnp.zeros_like(cnt_ref)

        dinv = lax.rsqrt(deg_ref[...] + 1.0)
        act = _elu(_load_groups(a_ref, NGI) * dinv + b_ref[...])
        ids = lax.broadcasted_iota(jnp.int32, (BM, NB), 1)
        oh = (bat_ref[...] == ids).astype(jnp.float32)
        dn = (((0,), (0,)), ((), ()))
        sums_ref[...] += lax.dot_general(oh, act, dn,
                                         preferred_element_type=jnp.float32)
        cnt_ref[...] += lax.dot_general(oh, jnp.ones((BM, 1), jnp.float32), dn,
                                        preferred_element_type=jnp.float32)

    return pl.pallas_call(
        body,
        grid=(N_PAD // BM,),
        in_specs=[
            pl.BlockSpec((NGI, BM, 128), lambda i: (0, i, 0)),
            pl.BlockSpec((1, F), lambda i: (0, 0)),
            pl.BlockSpec((BM, 1), lambda i: (i, 0)),
            pl.BlockSpec((BM, 1), lambda i: (i, 0)),
        ],
        out_specs=[
            pl.BlockSpec((NB, F), lambda i: (0, 0)),
            pl.BlockSpec((NB, 1), lambda i: (0, 0)),
        ],
        out_shape=[
            jax.ShapeDtypeStruct((NB, F), jnp.float32),
            jax.ShapeDtypeStruct((NB, 1), jnp.float32),
        ],
    )(A, b, deg, batch)


def _tc_head(sums_s, cnt_s, sums_p, cnt_p, dose, time,
             W_lx, b_lx, W_le, b_le, W_ld, b_ld, W_lt, b_lt,
             W1x, W1e, w1d, w1t, b_l1, W_l2p, b_l2p):
    OUTP = W_l2p.shape[1]

    def body(ss_ref, cs_ref, sp_ref, cp_ref, do_ref, ti_ref,
             wlx_ref, blx_ref, wle_ref, ble_ref, wld_ref, bld_ref,
             wlt_ref, blt_ref, w1x_ref, w1e_ref, w1d_ref, w1t_ref,
             bl1_ref, wl2_ref, bl2_ref, o_ref):
        xs = ss_ref[...] / jnp.maximum(cs_ref[...], 1.0)
        xp = sp_ref[...] / jnp.maximum(cp_ref[...], 1.0)
        x = jnp.dot(xs, wlx_ref[...], preferred_element_type=jnp.float32) \
            + blx_ref[...]
        e = jnp.dot(xp, wle_ref[...], preferred_element_type=jnp.float32) \
            + ble_ref[...]
        d = do_ref[...] * wld_ref[...] + bld_ref[...]
        t = ti_ref[...] * wlt_ref[...] + blt_ref[...]
        h = (jnp.dot(x, w1x_ref[...], preferred_element_type=jnp.float32)
             + jnp.dot(e, w1e_ref[...], preferred_element_type=jnp.float32)
             + jnp.dot(d, w1d_ref[...], preferred_element_type=jnp.float32)
             + jnp.dot(t, w1t_ref[...], preferred_element_type=jnp.float32)
             + bl1_ref[...])
        h = _elu(h)
        o_ref[...] = jnp.dot(h, wl2_ref[...],
                             preferred_element_type=jnp.float32) + bl2_ref[...]

    return pl.pallas_call(
        body,
        out_shape=jax.ShapeDtypeStruct((NB, OUTP), jnp.float32),
    )(sums_s, cnt_s, sums_p, cnt_p, dose, time,
      W_lx, b_lx, W_le, b_le, W_ld, b_ld, W_lt, b_lt,
      W1x, W1e, w1d, w1t, b_l1, W_l2p, b_l2p)


_MESH = dict(core_axis_name="c", subcore_axis_name="s")


def _sc_degree(dst_s2, dst_p2):
    mesh = plsc.VectorSubcoreMesh(**_MESH)

    @functools.partial(
        pl.kernel,
        out_type=(jax.ShapeDtypeStruct((N_PAD,), jnp.float32),
                  jax.ShapeDtypeStruct((N_PAD,), jnp.float32)),
        mesh=mesh,
        scratch_types=[
            pltpu.VMEM((CPT, CHUNK), jnp.int32),
            pltpu.VMEM((RPT,), jnp.float32),
            pltpu.VMEM((CHUNK,), jnp.float32),
            pltpu.VMEM_SHARED((N_PAD,), jnp.float32),
            pltpu.SemaphoreType.DMA,
        ],
    )
    def k(ds_ref, dp_ref, degs_ref, degp_ref, idx_v, zbuf, obuf, acc, sem):
        c = lax.axis_index("c")
        t = lax.axis_index("s")

        def fz(i, _):
            zbuf[pl.ds(i * 16, 16)] = jnp.zeros((16,), jnp.float32)
            return 0

        lax.fori_loop(0, RPT // 16, fz, 0)
        for i in range(CHUNK // 16):
            obuf[pl.ds(i * 16, 16)] = jnp.ones((16,), jnp.float32)
        pltpu.sync_copy(zbuf, acc.at[pl.ds(t * RPT, RPT)])

        for core in range(2):
            @pl.when(c == core)
            def _(core=core):
                din = ds_ref if core == 0 else dp_ref
                dout = degs_ref if core == 0 else degp_ref
                pltpu.sync_copy(din.at[t], idx_v)
                plsc.subcore_barrier()

                def fs(o, _):
                    descs = []
                    for b in range(8):
                        dst = acc.at[plsc.Indices(idx_v.at[o * 8 + b],
                                                  ignored_value=-1)]
                        descs.append(pltpu.async_copy(obuf, dst, sem,
                                                      add=True))
                    for d in descs:
                        d.wait()
                    return 0

                lax.fori_loop(0, CPT // 8, fs, 0)
                for i in range((CPT // 8) * 8, CPT):
                    dst = acc.at[plsc.Indices(idx_v.at[i], ignored_value=-1)]
                    pltpu.async_copy(obuf, dst, sem, add=True).wait()
                plsc.subcore_barrier()
                pltpu.sync_copy(acc.at[pl.ds(t * RPT, RPT)], zbuf)
                pltpu.sync_copy(zbuf, dout.at[pl.ds(t * RPT, RPT)])

    return k(dst_s2, dst_p2)


def _two_hop(n_chunks, nbuf, a_start, a_wait, b_start, b_wait):
    q = n_chunks // nbuf
    r = n_chunks - q * nbuf
    if q > 0:
        for b in range(nbuf):
            a_start(b, b)
        if q > 1:
            def step(j, _):
                base = j * nbuf
                for b in range(nbuf):
                    b_wait(base - nbuf + b, b)
                    a_start(base + b, b)
                for b in range(nbuf):
                    a_wait(base + b, b)
                    b_start(base + b, b)
                return 0

            for b in range(nbuf):
                a_wait(b, b)
                b_start(b, b)
            lax.fori_loop(1, q, step, 0)
        else:
            for b in range(nbuf):
                a_wait(b, b)
                b_start(b, b)
        for b in range(nbuf):
            b_wait((q - 1) * nbuf + b, b)
    for i in range(r):
        c = q * nbuf + i
        a_start(c, i)
        a_wait(c, i)
        b_start(c, i)
        b_wait(c, i)


def _three_hop(n_chunks, nbuf, a_start, a_wait, b_start, b_wait,
               c_start, c_wait):
    q = n_chunks // nbuf
    assert q >= 2 and n_chunks == q * nbuf

    def round_(j, first):
        base = j * nbuf
        for b in range(nbuf):
            if not first:
                c_wait(base - nbuf + b, b)
            a_start(base + b, b)
        for b in range(nbuf):
            a_wait(base + b, b)
            b_start(base + b, b)
        for b in range(nbuf):
            b_wait(base + b, b)
            c_start(base + b, b)

    round_(0, True)

    def step(j, _):
        round_(j, False)
        return 0

    lax.fori_loop(1, q, step, 0)
    for b in range(nbuf):
        c_wait((q - 1) * nbuf + b, b)


def _sc_agg(hn, packed):
    NG = hn.shape[0]
    mesh = plsc.VectorSubcoreMesh(**_MESH)

    @functools.partial(
        pl.kernel,
        out_type=jax.ShapeDtypeStruct((NG, N_PAD, 128), jnp.float32),
        mesh=mesh,
        scratch_types=[
            pltpu.VMEM((NBUF, CHUNK), jnp.int32),
            pltpu.VMEM((NBUF, CHUNK), jnp.int32),
            pltpu.VMEM((NBUF, CHUNK), jnp.int32),
            pltpu.VMEM((NBUF, CHUNK, 128), jnp.float32),
            pltpu.VMEM_SHARED((RNG, 128), jnp.float32),
        ] + [pltpu.SemaphoreType.DMA] * (3 * NBUF),
    )
    def k(hn_ref, pk_ref, out_ref, pkbuf, sbuf, dbuf, rows, acc, *sems):
        psem = sems[:NBUF]
        gsem = sems[NBUF:2 * NBUF]
        ssem = sems[2 * NBUF:]
        c = lax.axis_index("c")
        t = lax.axis_index("s")

        for ri in range(NRANGE // 2):
            r = c * (NRANGE // 2) + ri

            for g in range(NG):
                def rrange(i):
                    return pl.ds(r * RNG + t * RPR + i * WCH, WCH)

                def arange_(i):
                    return pl.ds(t * RPR + i * WCH, WCH)

                def wrows(b):
                    return rows.at[b].at[pl.ds(0, WCH)]

                def ia(i, b, wait=False):
                    d = pltpu.make_async_copy(hn_ref.at[g, rrange(i)],
                                              wrows(b), gsem[b])
                    d.wait() if wait else d.start()

                def ib(i, b, wait=False):
                    d = pltpu.make_async_copy(wrows(b), acc.at[arange_(i)],
                                              ssem[b])
                    d.wait() if wait else d.start()

                _two_hop(RPR // WCH, NBUF,
                         lambda i, b: ia(i, b), lambda i, b: ia(i, b, True),
                         lambda i, b: ib(i, b), lambda i, b: ib(i, b, True))
                plsc.subcore_barrier()

                def grows(b):
                    return rows.at[b]

                def pk_start(i, b):
                    pltpu.async_copy(pk_ref.at[r, t, i], pkbuf.at[b],
                                     psem[b])

                def pk_wait(i, b):
                    pltpu.make_async_copy(pk_ref.at[r, t, i], pkbuf.at[b],
                                          psem[b]).wait()

                def gather_start(i, b):
                    for kk in range(CHUNK // 16):
                        sl = pl.ds(kk * 16, 16)
                        w = pkbuf[b, sl]
                        sbuf[b, sl] = lax.bitwise_and(w, 0xFFFF)
                        dbuf[b, sl] = lax.shift_right_arithmetic(w, 16)
                    idx = plsc.Indices(sbuf.at[b], ignored_value=0xFFFF)
                    pltpu.async_copy(hn_ref.at[g].at[idx], grows(b), gsem[b])

                def gather_wait(i, b):
                    idx = plsc.Indices(sbuf.at[b], ignored_value=0xFFFF)
                    pltpu.make_async_copy(hn_ref.at[g].at[idx], grows(b),
                                          gsem[b]).wait()

                def scatter_start(i, b):
                    dst = acc.at[plsc.Indices(dbuf.at[b], ignored_value=-1)]
                    pltpu.async_copy(grows(b), dst, ssem[b], add=True)

                def scatter_wait(i, b):
                    dst = acc.at[plsc.Indices(dbuf.at[b], ignored_value=-1)]
                    pltpu.make_async_copy(grows(b), dst, ssem[b]).wait()

                _three_hop(CPT, NBUF, pk_start, pk_wait, gather_start,
                           gather_wait, scatter_start, scatter_wait)
                plsc.subcore_barrier()

                def wa(i, b, wait=False):
                    d = pltpu.make_async_copy(acc.at[arange_(i)], wrows(b),
                                              gsem[b])
                    d.wait() if wait else d.start()

                def wb(i, b, wait=False):
                    d = pltpu.make_async_copy(wrows(b),
                                              out_ref.at[g, rrange(i)],
                                              ssem[b])
                    d.wait() if wait else d.start()

                _two_hop(RPR // WCH, NBUF,
                         lambda i, b: wa(i, b), lambda i, b: wa(i, b, True),
                         lambda i, b: wb(i, b), lambda i, b: wb(i, b, True))
                plsc.subcore_barrier()

    return k(hn, packed)


def _prep_edges(edge_index):
    e = edge_index.shape[1]
    src = jnp.pad(edge_index[0], (0, E_PAD - e), constant_values=0)
    dst = jnp.pad(edge_index[1], (0, E_PAD - e), constant_values=-1)
    rid = jnp.arange(NRANGE, dtype=jnp.int32)[:, None]
    in_rng = (dst[None, :] >= rid * RNG) & (dst[None, :] < (rid + 1) * RNG)
    word = jnp.where(in_rng,
                     ((dst[None, :] - rid * RNG) << 16) | src[None, :],
                     jnp.int32(-1))
    return (dst.reshape(16, CPT, CHUNK),
            word.reshape(NRANGE, 16, CPT, CHUNK))


def _tower(x, packed, deg, W1, b1, W2, b2, batch):
    hn = _tc_first(x, W1, deg)
    A = _sc_agg(hn, packed)
    hn = _tc_mid(A, W2[0], b1.reshape(1, -1), deg)
    A = _sc_agg(hn, packed)
    hn = _tc_mid(A, W2[1], b2[0].reshape(1, -1), deg)
    A = _sc_agg(hn, packed)
    return _tc_pool(A, b2[1].reshape(1, -1), deg, batch)


def kernel(x_s, edge_index_s, x_p, edge_index_p, x_s_batch, x_p_batch,
           dose, time, drop_pert,
           W_c1, b_c1, W_c2, b_c2, W_g1, b_g1, W_g2, b_g2,
           W_lx, b_lx, W_le, b_le, W_ld, b_ld, W_lt, b_lt,
           W_l1, b_l1, W_l2, b_l2):
    n_s = x_s.shape[0]
    n_p = x_p.shape[0]
    xs = jnp.pad(x_s, ((0, N_PAD - n_s), (0, 0)))
    xp = jnp.pad(x_p, ((0, N_PAD - n_p), (0, 6)))
    Wg1 = jnp.pad(W_g1, ((0, 6), (0, 0)))
    bat_s = jnp.pad(x_s_batch, (0, N_PAD - n_s),
                    constant_values=NB).reshape(N_PAD, 1)
    bat_p = jnp.pad(x_p_batch, (0, N_PAD - n_p),
                    constant_values=NB).reshape(N_PAD, 1)
    dst_s2, packed_s = _prep_edges(edge_index_s)
    dst_p2, packed_p = _prep_edges(edge_index_p)

    deg_s, deg_p = _sc_degree(dst_s2, dst_p2)
    deg_s = deg_s.reshape(N_PAD, 1)
    deg_p = deg_p.reshape(N_PAD, 1)

    sums_s, cnt_s = _tower(xs, packed_s, deg_s, W_c1, b_c1, W_c2, b_c2,
                           bat_s)
    sums_p, cnt_p = _tower(xp, packed_p, deg_p, Wg1, b_g1, W_g2, b_g2,
                           bat_p)

    OUT = W_l2.shape[1]
    OUTP = ((OUT + 127) // 128) * 128
    W_l2p = jnp.pad(W_l2, ((0, 0), (0, OUTP - OUT)))
    b_l2p = jnp.pad(b_l2, (0, OUTP - OUT)).reshape(1, OUTP)

    D = W_lx.shape[1]
    out = _tc_head(sums_s, cnt_s, sums_p, cnt_p, dose, time,
                   W_lx, b_lx.reshape(1, -1), W_le, b_le.reshape(1, -1),
                   W_ld, b_ld.reshape(1, 1), W_lt, b_lt.reshape(1, 1),
                   W_l1[:D], W_l1[D:2 * D], W_l1[2 * D:2 * D + 1],
                   W_l1[2 * D + 1:2 * D + 2], b_l1.reshape(1, -1),
                   W_l2p, b_l2p)
    return out[:, :OUT]


# speedup vs baseline: 4.3125x; 1.0819x over previous
import functools

import jax
import jax.numpy as jnp
from jax import lax
from jax.experimental import pallas as pl
from jax.experimental.pallas import tpu as pltpu
from jax.experimental.pallas import tpu_sc as plsc

def _elu(x):
    return jnp.where(x > 0, x, jnp.exp(jnp.minimum(x, 0.0)) - 1.0)


N_PAD = 51200
RPT = N_PAD // 16
BM = 512
CHUNK = 112
CPT = 448
E_PAD = 16 * CPT * CHUNK
NB = 64
NRANGE = 8
RNG = N_PAD // NRANGE
RPR = RNG // 16
WCH = 80
NBUF = 4


def _store_groups(o_ref, h, ng):
    for g in range(ng):
        o_ref[g] = h[:, g * 128:(g + 1) * 128]


def _load_groups(a_ref, ng):
    if ng == 1:
        return a_ref[0]
    return jnp.concatenate([a_ref[g] for g in range(ng)], axis=1)


def _tc_first(x, W, deg):
    F = W.shape[1]
    NG = F // 128

    def body(x_ref, w_ref, deg_ref, o_ref):
        dinv = lax.rsqrt(deg_ref[...] + 1.0)
        h = jnp.dot(x_ref[...], w_ref[...], preferred_element_type=jnp.float32)
        _store_groups(o_ref, h * dinv, NG)

    return pl.pallas_call(
        body,
        grid=(N_PAD // BM,),
        in_specs=[
            pl.BlockSpec((BM, x.shape[1]), lambda i: (i, 0)),
            pl.BlockSpec(W.shape, lambda i: (0, 0)),
            pl.BlockSpec((BM, 1), lambda i: (i, 0)),
        ],
        out_specs=pl.BlockSpec((NG, BM, 128), lambda i: (0, i, 0)),
        out_shape=jax.ShapeDtypeStruct((NG, N_PAD, 128), jnp.float32),
    )(x, W, deg)


def _tc_mid(A, W, b, deg):
    FI = W.shape[0]
    F = W.shape[1]
    NGI = FI // 128
    NG = F // 128

    def body(a_ref, w_ref, b_ref, deg_ref, o_ref):
        dinv = lax.rsqrt(deg_ref[...] + 1.0)
        act = _elu(_load_groups(a_ref, NGI) * dinv + b_ref[...])
        h = jnp.dot(act, w_ref[...], preferred_element_type=jnp.float32)
        _store_groups(o_ref, h * dinv, NG)

    return pl.pallas_call(
        body,
        grid=(N_PAD // BM,),
        in_specs=[
            pl.BlockSpec((NGI, BM, 128), lambda i: (0, i, 0)),
            pl.BlockSpec(W.shape, lambda i: (0, 0)),
            pl.BlockSpec((1, F), lambda i: (0, 0)),
            pl.BlockSpec((BM, 1), lambda i: (i, 0)),
        ],
        out_specs=pl.BlockSpec((NG, BM, 128), lambda i: (0, i, 0)),
        out_shape=jax.ShapeDtypeStruct((NG, N_PAD, 128), jnp.float32),
    )(A, W, b, deg)


def _tc_pool(A, b, deg, batch):
    NGI = A.shape[0]
    F = NGI * 128

    def body(a_ref, b_ref, deg_ref, bat_ref, sums_ref, cnt_ref):
        i = pl.program_id(0)

        @pl.when(i == 0)
        def _():
            sums_ref[...] = jnp.zeros_like(sums_ref)
            cnt_ref[...] = jnp.zeros_like(cnt_ref)

        dinv = lax.rsqrt(deg_ref[...] + 1.0)
        act = _elu(_load_groups(a_ref, NGI) * dinv + b_ref[...])
        ids = lax.broadcasted_iota(jnp.int32, (BM, NB), 1)
        oh = (bat_ref[...] == ids).astype(jnp.float32)
        dn = (((0,), (0,)), ((), ()))
        sums_ref[...] += lax.dot_general(oh, act, dn,
                                         preferred_element_type=jnp.float32)
        cnt_ref[...] += lax.dot_general(oh, jnp.ones((BM, 1), jnp.float32), dn,
                                        preferred_element_type=jnp.float32)

    return pl.pallas_call(
        body,
        grid=(N_PAD // BM,),
        in_specs=[
            pl.BlockSpec((NGI, BM, 128), lambda i: (0, i, 0)),
            pl.BlockSpec((1, F), lambda i: (0, 0)),
            pl.BlockSpec((BM, 1), lambda i: (i, 0)),
            pl.BlockSpec((BM, 1), lambda i: (i, 0)),
        ],
        out_specs=[
            pl.BlockSpec((NB, F), lambda i: (0, 0)),
            pl.BlockSpec((NB, 1), lambda i: (0, 0)),
        ],
        out_shape=[
            jax.ShapeDtypeStruct((NB, F), jnp.float32),
            jax.ShapeDtypeStruct((NB, 1), jnp.float32),
        ],
    )(A, b, deg, batch)


def _tc_head(sums_s, cnt_s, sums_p, cnt_p, dose, time,
             W_lx, b_lx, W_le, b_le, W_ld, b_ld, W_lt, b_lt,
             W1x, W1e, w1d, w1t, b_l1, W_l2p, b_l2p):
    OUTP = W_l2p.shape[1]

    def body(ss_ref, cs_ref, sp_ref, cp_ref, do_ref, ti_ref,
             wlx_ref, blx_ref, wle_ref, ble_ref, wld_ref, bld_ref,
             wlt_ref, blt_ref, w1x_ref, w1e_ref, w1d_ref, w1t_ref,
             bl1_ref, wl2_ref, bl2_ref, o_ref):
        xs = ss_ref[...] / jnp.maximum(cs_ref[...], 1.0)
        xp = sp_ref[...] / jnp.maximum(cp_ref[...], 1.0)
        x = jnp.dot(xs, wlx_ref[...], preferred_element_type=jnp.float32) \
            + blx_ref[...]
        e = jnp.dot(xp, wle_ref[...], preferred_element_type=jnp.float32) \
            + ble_ref[...]
        d = do_ref[...] * wld_ref[...] + bld_ref[...]
        t = ti_ref[...] * wlt_ref[...] + blt_ref[...]
        h = (jnp.dot(x, w1x_ref[...], preferred_element_type=jnp.float32)
             + jnp.dot(e, w1e_ref[...], preferred_element_type=jnp.float32)
             + jnp.dot(d, w1d_ref[...], preferred_element_type=jnp.float32)
             + jnp.dot(t, w1t_ref[...], preferred_element_type=jnp.float32)
             + bl1_ref[...])
        h = _elu(h)
        o_ref[...] = jnp.dot(h, wl2_ref[...],
                             preferred_element_type=jnp.float32) + bl2_ref[...]

    return pl.pallas_call(
        body,
        out_shape=jax.ShapeDtypeStruct((NB, OUTP), jnp.float32),
    )(sums_s, cnt_s, sums_p, cnt_p, dose, time,
      W_lx, b_lx, W_le, b_le, W_ld, b_ld, W_lt, b_lt,
      W1x, W1e, w1d, w1t, b_l1, W_l2p, b_l2p)


_MESH = dict(core_axis_name="c", subcore_axis_name="s")


def _sc_degree(dst_s2, dst_p2):
    mesh = plsc.VectorSubcoreMesh(**_MESH)

    @functools.partial(
        pl.kernel,
        out_type=(jax.ShapeDtypeStruct((N_PAD,), jnp.float32),
                  jax.ShapeDtypeStruct((N_PAD,), jnp.float32)),
        mesh=mesh,
        scratch_types=[
            pltpu.VMEM((CPT, CHUNK), jnp.int32),
            pltpu.VMEM((RPT,), jnp.float32),
            pltpu.VMEM((CHUNK,), jnp.float32),
            pltpu.VMEM_SHARED((N_PAD,), jnp.float32),
            pltpu.SemaphoreType.DMA,
        ],
    )
    def k(ds_ref, dp_ref, degs_ref, degp_ref, idx_v, zbuf, obuf, acc, sem):
        c = lax.axis_index("c")
        t = lax.axis_index("s")

        def fz(i, _):
            zbuf[pl.ds(i * 16, 16)] = jnp.zeros((16,), jnp.float32)
            return 0

        lax.fori_loop(0, RPT // 16, fz, 0)
        for i in range(CHUNK // 16):
            obuf[pl.ds(i * 16, 16)] = jnp.ones((16,), jnp.float32)
        pltpu.sync_copy(zbuf, acc.at[pl.ds(t * RPT, RPT)])

        for core in range(2):
            @pl.when(c == core)
            def _(core=core):
                din = ds_ref if core == 0 else dp_ref
                dout = degs_ref if core == 0 else degp_ref
                pltpu.sync_copy(din.at[t], idx_v)
                plsc.subcore_barrier()

                def fs(o, _):
                    descs = []
                    for b in range(8):
                        dst = acc.at[plsc.Indices(idx_v.at[o * 8 + b],
                                                  ignored_value=-1)]
                        descs.append(pltpu.async_copy(obuf, dst, sem,
                                                      add=True))
                    for d in descs:
                        d.wait()
                    return 0

                lax.fori_loop(0, CPT // 8, fs, 0)
                for i in range((CPT // 8) * 8, CPT):
                    dst = acc.at[plsc.Indices(idx_v.at[i], ignored_value=-1)]
                    pltpu.async_copy(obuf, dst, sem, add=True).wait()
                plsc.subcore_barrier()
                pltpu.sync_copy(acc.at[pl.ds(t * RPT, RPT)], zbuf)
                pltpu.sync_copy(zbuf, dout.at[pl.ds(t * RPT, RPT)])

    return k(dst_s2, dst_p2)


def _two_hop(n_chunks, nbuf, a_start, a_wait, b_start, b_wait):
    q = n_chunks // nbuf
    r = n_chunks - q * nbuf
    if q > 0:
        for b in range(nbuf):
            a_start(b, b)
        if q > 1:
            def step(j, _):
                base = j * nbuf
                for b in range(nbuf):
                    b_wait(base - nbuf + b, b)
                    a_start(base + b, b)
                for b in range(nbuf):
                    a_wait(base + b, b)
                    b_start(base + b, b)
                return 0

            for b in range(nbuf):
                a_wait(b, b)
                b_start(b, b)
            lax.fori_loop(1, q, step, 0)
        else:
            for b in range(nbuf):
                a_wait(b, b)
                b_start(b, b)
        for b in range(nbuf):
            b_wait((q - 1) * nbuf + b, b)
    for i in range(r):
        c = q * nbuf + i
        a_start(c, i)
        a_wait(c, i)
        b_start(c, i)
        b_wait(c, i)


def _three_hop(n_chunks, nbuf, a_start, a_wait, b_start, b_wait,
               c_start, c_wait):
    q = n_chunks // nbuf
    assert q >= 2 and n_chunks == q * nbuf

    def round_(j, first):
        base = j * nbuf
        for b in range(nbuf):
            if not first:
                c_wait(base - nbuf + b, b)
            a_start(base + b, b)
        for b in range(nbuf):
            a_wait(base + b, b)
            b_start(base + b, b)
        for b in range(nbuf):
            b_wait(base + b, b)
            c_start(base + b, b)

    round_(0, True)

    def step(j, _):
        round_(j, False)
        return 0

    lax.fori_loop(1, q, step, 0)
    for b in range(nbuf):
        c_wait((q - 1) * nbuf + b, b)


def _sc_agg(hn, packed):
    NG = hn.shape[0]
    mesh = plsc.VectorSubcoreMesh(**_MESH)

    @functools.partial(
        pl.kernel,
        out_type=jax.ShapeDtypeStruct((NG, N_PAD, 128), jnp.float32),
        mesh=mesh,
        scratch_types=[
            pltpu.VMEM((NBUF, CHUNK), jnp.int32),
            pltpu.VMEM((NBUF, CHUNK), jnp.int32),
            pltpu.VMEM((NBUF, CHUNK), jnp.int32),
            pltpu.VMEM((NBUF, CHUNK, 128), jnp.float32),
            pltpu.VMEM_SHARED((RNG, 128), jnp.float32),
        ] + [pltpu.SemaphoreType.DMA] * (3 * NBUF),
    )
    def k(hn_ref, pk_ref, out_ref, pkbuf, sbuf, dbuf, rows, acc, *sems):
        psem = sems[:NBUF]
        gsem = sems[NBUF:2 * NBUF]
        ssem = sems[2 * NBUF:]
        c = lax.axis_index("c")
        t = lax.axis_index("s")

        for ri in range(NRANGE // 2):
            r = c * (NRANGE // 2) + ri

            for g in range(NG):
                def rrange(i):
                    return pl.ds(r * RNG + t * RPR + i * WCH, WCH)

                def arange_(i):
                    return pl.ds(t * RPR + i * WCH, WCH)

                def wrows(b):
                    return rows.at[b].at[pl.ds(0, WCH)]

                def ia(i, b, wait=False):
                    d = pltpu.make_async_copy(hn_ref.at[g, rrange(i)],
                                              wrows(b), gsem[b])
                    d.wait() if wait else d.start()

                def ib(i, b, wait=False):
                    d = pltpu.make_async_copy(wrows(b), acc.at[arange_(i)],
                                              ssem[b])
                    d.wait() if wait else d.start()

                _two_hop(RPR // WCH, NBUF,
                         lambda i, b: ia(i, b), lambda i, b: ia(i, b, True),
                         lambda i, b: ib(i, b), lambda i, b: ib(i, b, True))
                plsc.subcore_barrier()

                def grows(b):
                    return rows.at[b]

                def pk_start(i, b):
                    pltpu.async_copy(pk_ref.at[r, t, i], pkbuf.at[b],
                                     psem[b])

                def pk_wait(i, b):
                    pltpu.make_async_copy(pk_ref.at[r, t, i], pkbuf.at[b],
                                          psem[b]).wait()

                def gather_start(i, b):
                    for kk in range(CHUNK // 16):
                        sl = pl.ds(kk * 16, 16)
                        w = pkbuf[b, sl]
                        sbuf[b, sl] = lax.bitwise_and(w, 0xFFFF)
                        dbuf[b, sl] = lax.shift_right_arithmetic(w, 16)
                    idx = plsc.Indices(sbuf.at[b], ignored_value=0xFFFF)
                    pltpu.async_copy(hn_ref.at[g].at[idx], grows(b), gsem[b])

                def gather_wait(i, b):
                    idx = plsc.Indices(sbuf.at[b], ignored_value=0xFFFF)
                    pltpu.make_async_copy(hn_ref.at[g].at[idx], grows(b),
                                          gsem[b]).wait()

                def scatter_start(i, b):
                    dst = acc.at[plsc.Indices(dbuf.at[b], ignored_value=-1)]
                    pltpu.async_copy(grows(b), dst, ssem[b], add=True)

                def scatter_wait(i, b):
                    dst = acc.at[plsc.Indices(dbuf.at[b], ignored_value=-1)]
                    pltpu.make_async_copy(grows(b), dst, ssem[b]).wait()

                _three_hop(CPT, NBUF, pk_start, pk_wait, gather_start,
                           gather_wait, scatter_start, scatter_wait)
                plsc.subcore_barrier()

                def wa(i, b, wait=False):
                    d = pltpu.make_async_copy(acc.at[arange_(i)], wrows(b),
                                              gsem[b])
                    d.wait() if wait else d.start()

                def wb(i, b, wait=False):
                    d = pltpu.make_async_copy(wrows(b),
                                              out_ref.at[g, rrange(i)],
                                              ssem[b])
                    d.wait() if wait else d.start()

                _two_hop(RPR // WCH, NBUF,
                         lambda i, b: wa(i, b), lambda i, b: wa(i, b, True),
                         lambda i, b: wb(i, b), lambda i, b: wb(i, b, True))
                plsc.subcore_barrier()

    return k(hn, packed)


def _prep_edges(edge_index):
    e = edge_index.shape[1]
    src = jnp.pad(edge_index[0], (0, E_PAD - e), constant_values=0)
    dst = jnp.pad(edge_index[1], (0, E_PAD - e), constant_values=-1)
    rid = jnp.arange(NRANGE, dtype=jnp.int32)[:, None]
    in_rng = (dst[None, :] >= rid * RNG) & (dst[None, :] < (rid + 1) * RNG)
    word = jnp.where(in_rng,
                     ((dst[None, :] - rid * RNG) << 16) | src[None, :],
                     jnp.int32(-1))
    return (dst.reshape(16, CPT, CHUNK),
            word.reshape(NRANGE, 16, CPT, CHUNK))


def _tower(x, packed, deg, W1, b1, W2, b2, batch):
    hn = _tc_first(x, W1, deg)
    A = _sc_agg(hn, packed)
    hn = _tc_mid(A, W2[0], b1.reshape(1, -1), deg)
    A = _sc_agg(hn, packed)
    hn = _tc_mid(A, W2[1], b2[0].reshape(1, -1), deg)
    A = _sc_agg(hn, packed)
    return _tc_pool(A, b2[1].reshape(1, -1), deg, batch)


def kernel(x_s, edge_index_s, x_p, edge_index_p, x_s_batch, x_p_batch,
           dose, time, drop_pert,
           W_c1, b_c1, W_c2, b_c2, W_g1, b_g1, W_g2, b_g2,
           W_lx, b_lx, W_le, b_le, W_ld, b_ld, W_lt, b_lt,
           W_l1, b_l1, W_l2, b_l2):
    n_s = x_s.shape[0]
    n_p = x_p.shape[0]
    xs = jnp.pad(x_s, ((0, N_PAD - n_s), (0, 0)))
    xp = jnp.pad(x_p, ((0, N_PAD - n_p), (0, 6)))
    Wg1 = jnp.pad(W_g1, ((0, 6), (0, 0)))
    bat_s = jnp.pad(x_s_batch, (0, N_PAD - n_s),
                    constant_values=NB).reshape(N_PAD, 1)
    bat_p = jnp.pad(x_p_batch, (0, N_PAD - n_p),
                    constant_values=NB).reshape(N_PAD, 1)
    dst_s2, packed_s = _prep_edges(edge_index_s)
    dst_p2, packed_p = _prep_edges(edge_index_p)

    deg_s, deg_p = _sc_degree(dst_s2, dst_p2)
    deg_s = deg_s.reshape(N_PAD, 1)
    deg_p = deg_p.reshape(N_PAD, 1)

    sums_s, cnt_s = _tower(xs, packed_s, deg_s, W_c1, b_c1, W_c2, b_c2,
                           bat_s)
    sums_p, cnt_p = _tower(xp, packed_p, deg_p, Wg1, b_g1, W_g2, b_g2,
                           bat_p)

    OUT = W_l2.shape[1]
    OUTP = ((OUT + 127) // 128) * 128
    W_l2p = jnp.pad(W_l2, ((0, 0), (0, OUTP - OUT)))
    b_l2p = jnp.pad(b_l2, (0, OUTP - OUT)).reshape(1, OUTP)

    D = W_lx.shape[1]
    out = _tc_head(sums_s, cnt_s, sums_p, cnt_p, dose, time,
                   W_lx, b_lx.reshape(1, -1), W_le, b_le.reshape(1, -1),
                   W_ld, b_ld.reshape(1, 1), W_lt, b_lt.reshape(1, 1),
                   W_l1[:D], W_l1[D:2 * D], W_l1[2 * D:2 * D + 1],
                   W_l1[2 * D + 1:2 * D + 2], b_l1.reshape(1, -1),
                   W_l2p, b_l2p)
    return out[:, :OUT]


# speedup vs baseline: 6.1177x; 1.4186x over previous
import functools

import jax
import jax.numpy as jnp
from jax import lax
from jax.experimental import pallas as pl
from jax.experimental.pallas import tpu as pltpu
from jax.experimental.pallas import tpu_sc as plsc

def _elu(x):
    return jnp.where(x > 0, x, jnp.exp(jnp.minimum(x, 0.0)) - 1.0)


N_PAD = 51200
RPT = N_PAD // 16
BM = 512
CHUNK = 112
CPT = 448
E_PAD = 16 * CPT * CHUNK
NB = 64
NRANGE = 8
RNG = N_PAD // NRANGE
RPR = RNG // 16
WCH = 80
NBUF = 4


def _store_groups(o_ref, h, ng):
    for g in range(ng):
        o_ref[g] = h[:, g * 128:(g + 1) * 128]


def _load_groups(a_ref, ng):
    if ng == 1:
        return a_ref[0]
    return jnp.concatenate([a_ref[g] for g in range(ng)], axis=1)


def _tc_first(x, W, deg):
    F = W.shape[1]
    NG = F // 128

    def body(x_ref, w_ref, deg_ref, o_ref):
        dinv = lax.rsqrt(deg_ref[...] + 1.0)
        h = jnp.dot(x_ref[...], w_ref[...], preferred_element_type=jnp.float32)
        _store_groups(o_ref, h * dinv, NG)

    return pl.pallas_call(
        body,
        grid=(N_PAD // BM,),
        in_specs=[
            pl.BlockSpec((BM, x.shape[1]), lambda i: (i, 0)),
            pl.BlockSpec(W.shape, lambda i: (0, 0)),
            pl.BlockSpec((BM, 1), lambda i: (i, 0)),
        ],
        out_specs=pl.BlockSpec((NG, BM, 128), lambda i: (0, i, 0)),
        out_shape=jax.ShapeDtypeStruct((NG, N_PAD, 128), jnp.float32),
    )(x, W, deg)


def _tc_mid(A, W, b, deg):
    FI = W.shape[0]
    F = W.shape[1]
    NGI = FI // 128
    NG = F // 128

    def body(a_ref, w_ref, b_ref, deg_ref, o_ref):
        dinv = lax.rsqrt(deg_ref[...] + 1.0)
        act = _elu(_load_groups(a_ref, NGI) * dinv + b_ref[...])
        h = jnp.dot(act, w_ref[...], preferred_element_type=jnp.float32)
        _store_groups(o_ref, h * dinv, NG)

    return pl.pallas_call(
        body,
        grid=(N_PAD // BM,),
        in_specs=[
            pl.BlockSpec((NGI, BM, 128), lambda i: (0, i, 0)),
            pl.BlockSpec(W.shape, lambda i: (0, 0)),
            pl.BlockSpec((1, F), lambda i: (0, 0)),
            pl.BlockSpec((BM, 1), lambda i: (i, 0)),
        ],
        out_specs=pl.BlockSpec((NG, BM, 128), lambda i: (0, i, 0)),
        out_shape=jax.ShapeDtypeStruct((NG, N_PAD, 128), jnp.float32),
    )(A, W, b, deg)


def _tc_pool(A, b, deg, batch):
    NGI = A.shape[0]
    F = NGI * 128

    def body(a_ref, b_ref, deg_ref, bat_ref, sums_ref, cnt_ref):
        i = pl.program_id(0)

        @pl.when(i == 0)
        def _():
            sums_ref[...] = jnp.zeros_like(sums_ref)
            cnt_ref[...] = jnp.zeros_like(cnt_ref)

        dinv = lax.rsqrt(deg_ref[...] + 1.0)
        act = _elu(_load_groups(a_ref, NGI) * dinv + b_ref[...])
        ids = lax.broadcasted_iota(jnp.int32, (BM, NB), 1)
        oh = (bat_ref[...] == ids).astype(jnp.float32)
        dn = (((0,), (0,)), ((), ()))
        sums_ref[...] += lax.dot_general(oh, act, dn,
                                         preferred_element_type=jnp.float32)
        cnt_ref[...] += lax.dot_general(oh, jnp.ones((BM, 1), jnp.float32), dn,
                                        preferred_element_type=jnp.float32)

    return pl.pallas_call(
        body,
        grid=(N_PAD // BM,),
        in_specs=[
            pl.BlockSpec((NGI, BM, 128), lambda i: (0, i, 0)),
            pl.BlockSpec((1, F), lambda i: (0, 0)),
            pl.BlockSpec((BM, 1), lambda i: (i, 0)),
            pl.BlockSpec((BM, 1), lambda i: (i, 0)),
        ],
        out_specs=[
            pl.BlockSpec((NB, F), lambda i: (0, 0)),
            pl.BlockSpec((NB, 1), lambda i: (0, 0)),
        ],
        out_shape=[
            jax.ShapeDtypeStruct((NB, F), jnp.float32),
            jax.ShapeDtypeStruct((NB, 1), jnp.float32),
        ],
    )(A, b, deg, batch)


def _tc_head(sums_s, cnt_s, sums_p, cnt_p, dose, time,
             W_lx, b_lx, W_le, b_le, W_ld, b_ld, W_lt, b_lt,
             W1x, W1e, w1d, w1t, b_l1, W_l2p, b_l2p):
    OUTP = W_l2p.shape[1]

    def body(ss_ref, cs_ref, sp_ref, cp_ref, do_ref, ti_ref,
             wlx_ref, blx_ref, wle_ref, ble_ref, wld_ref, bld_ref,
             wlt_ref, blt_ref, w1x_ref, w1e_ref, w1d_ref, w1t_ref,
             bl1_ref, wl2_ref, bl2_ref, o_ref):
        xs = ss_ref[...] / jnp.maximum(cs_ref[...], 1.0)
        xp = sp_ref[...] / jnp.maximum(cp_ref[...], 1.0)
        x = jnp.dot(xs, wlx_ref[...], preferred_element_type=jnp.float32) \
            + blx_ref[...]
        e = jnp.dot(xp, wle_ref[...], preferred_element_type=jnp.float32) \
            + ble_ref[...]
        d = do_ref[...] * wld_ref[...] + bld_ref[...]
        t = ti_ref[...] * wlt_ref[...] + blt_ref[...]
        h = (jnp.dot(x, w1x_ref[...], preferred_element_type=jnp.float32)
             + jnp.dot(e, w1e_ref[...], preferred_element_type=jnp.float32)
             + jnp.dot(d, w1d_ref[...], preferred_element_type=jnp.float32)
             + jnp.dot(t, w1t_ref[...], preferred_element_type=jnp.float32)
             + bl1_ref[...])
        h = _elu(h)
        o_ref[...] = jnp.dot(h, wl2_ref[...],
                             preferred_element_type=jnp.float32) + bl2_ref[...]

    return pl.pallas_call(
        body,
        out_shape=jax.ShapeDtypeStruct((NB, OUTP), jnp.float32),
    )(sums_s, cnt_s, sums_p, cnt_p, dose, time,
      W_lx, b_lx, W_le, b_le, W_ld, b_ld, W_lt, b_lt,
      W1x, W1e, w1d, w1t, b_l1, W_l2p, b_l2p)


_MESH = dict(core_axis_name="c", subcore_axis_name="s")


def _sc_degree(dst_s2, dst_p2):
    mesh = plsc.VectorSubcoreMesh(**_MESH)

    @functools.partial(
        pl.kernel,
        out_type=(jax.ShapeDtypeStruct((N_PAD,), jnp.float32),
                  jax.ShapeDtypeStruct((N_PAD,), jnp.float32)),
        mesh=mesh,
        scratch_types=[
            pltpu.VMEM((CPT, CHUNK), jnp.int32),
            pltpu.VMEM((RPT,), jnp.float32),
            pltpu.VMEM((CHUNK,), jnp.float32),
            pltpu.VMEM_SHARED((N_PAD,), jnp.float32),
            pltpu.SemaphoreType.DMA,
        ],
    )
    def k(ds_ref, dp_ref, degs_ref, degp_ref, idx_v, zbuf, obuf, acc, sem):
        c = lax.axis_index("c")
        t = lax.axis_index("s")

        def fz(i, _):
            zbuf[pl.ds(i * 16, 16)] = jnp.zeros((16,), jnp.float32)
            return 0

        lax.fori_loop(0, RPT // 16, fz, 0)
        for i in range(CHUNK // 16):
            obuf[pl.ds(i * 16, 16)] = jnp.ones((16,), jnp.float32)
        pltpu.sync_copy(zbuf, acc.at[pl.ds(t * RPT, RPT)])

        for core in range(2):
            @pl.when(c == core)
            def _(core=core):
                din = ds_ref if core == 0 else dp_ref
                dout = degs_ref if core == 0 else degp_ref
                pltpu.sync_copy(din.at[t], idx_v)
                plsc.subcore_barrier()

                def fs(o, _):
                    descs = []
                    for b in range(8):
                        dst = acc.at[plsc.Indices(idx_v.at[o * 8 + b],
                                                  ignored_value=-1)]
                        descs.append(pltpu.async_copy(obuf, dst, sem,
                                                      add=True))
                    for d in descs:
                        d.wait()
                    return 0

                lax.fori_loop(0, CPT // 8, fs, 0)
                for i in range((CPT // 8) * 8, CPT):
                    dst = acc.at[plsc.Indices(idx_v.at[i], ignored_value=-1)]
                    pltpu.async_copy(obuf, dst, sem, add=True).wait()
                plsc.subcore_barrier()
                pltpu.sync_copy(acc.at[pl.ds(t * RPT, RPT)], zbuf)
                pltpu.sync_copy(zbuf, dout.at[pl.ds(t * RPT, RPT)])

    return k(dst_s2, dst_p2)


def _two_hop(n_chunks, nbuf, a_start, a_wait, b_start, b_wait):
    q = n_chunks // nbuf
    r = n_chunks - q * nbuf
    if q > 0:
        for b in range(nbuf):
            a_start(b, b)
        if q > 1:
            def step(j, _):
                base = j * nbuf
                for b in range(nbuf):
                    b_wait(base - nbuf + b, b)
                    a_start(base + b, b)
                for b in range(nbuf):
                    a_wait(base + b, b)
                    b_start(base + b, b)
                return 0

            for b in range(nbuf):
                a_wait(b, b)
                b_start(b, b)
            lax.fori_loop(1, q, step, 0)
        else:
            for b in range(nbuf):
                a_wait(b, b)
                b_start(b, b)
        for b in range(nbuf):
            b_wait((q - 1) * nbuf + b, b)
    for i in range(r):
        c = q * nbuf + i
        a_start(c, i)
        a_wait(c, i)
        b_start(c, i)
        b_wait(c, i)


def _three_hop(n_chunks, nbuf, a_start, a_wait, b_start, b_wait,
               c_start, c_wait):
    q = n_chunks // nbuf
    assert q >= 2 and n_chunks == q * nbuf

    def round_(j, first):
        base = j * nbuf
        for b in range(nbuf):
            if not first:
                c_wait(base - nbuf + b, b)
            a_start(base + b, b)
        for b in range(nbuf):
            a_wait(base + b, b)
            b_start(base + b, b)
        for b in range(nbuf):
            b_wait(base + b, b)
            c_start(base + b, b)

    round_(0, True)

    def step(j, _):
        round_(j, False)
        return 0

    lax.fori_loop(1, q, step, 0)
    for b in range(nbuf):
        c_wait((q - 1) * nbuf + b, b)


def _sc_agg(hn, packed):
    NG = hn.shape[0]
    mesh = plsc.VectorSubcoreMesh(**_MESH)
    PB = 2 * NBUF

    @functools.partial(
        pl.kernel,
        out_type=jax.ShapeDtypeStruct((NG, N_PAD, 128), jnp.float32),
        mesh=mesh,
        scratch_types=[
            pltpu.VMEM((PB, CHUNK), jnp.int32),
            pltpu.VMEM((NBUF, CHUNK), jnp.int32),
            pltpu.VMEM((NBUF, CHUNK), jnp.int32),
            pltpu.VMEM((NBUF, CHUNK, 128), jnp.float32),
            pltpu.VMEM_SHARED((RNG, 128), jnp.float32),
        ] + [pltpu.SemaphoreType.DMA] * (PB + 2 * NBUF),
    )
    def k(hn_ref, pk_ref, out_ref, pkbuf, sbuf, dbuf, rows, acc, *sems):
        psem = sems[:PB]
        gsem = sems[PB:PB + NBUF]
        ssem = sems[PB + NBUF:]
        c = lax.axis_index("c")
        t = lax.axis_index("s")

        for ri in range(NRANGE // 2):
            r = c * (NRANGE // 2) + ri

            for g in range(NG):
                def rrange(i):
                    return pl.ds(r * RNG + t * RPR + i * WCH, WCH)

                def arange_(i):
                    return pl.ds(t * RPR + i * WCH, WCH)

                def wrows(b):
                    return rows.at[b].at[pl.ds(0, WCH)]

                def ia(i, b, wait=False):
                    d = pltpu.make_async_copy(hn_ref.at[g, rrange(i)],
                                              wrows(b), gsem[b])
                    d.wait() if wait else d.start()

                def ib(i, b, wait=False):
                    d = pltpu.make_async_copy(wrows(b), acc.at[arange_(i)],
                                              ssem[b])
                    d.wait() if wait else d.start()

                _two_hop(RPR // WCH, NBUF,
                         lambda i, b: ia(i, b), lambda i, b: ia(i, b, True),
                         lambda i, b: ib(i, b), lambda i, b: ib(i, b, True))
                plsc.subcore_barrier()

                def grows(b):
                    return rows.at[b]

                def pk_start(i, s):
                    pltpu.async_copy(pk_ref.at[r, t, i], pkbuf.at[s],
                                     psem[s])

                def pk_wait(i, s):
                    pltpu.make_async_copy(pk_ref.at[r, t, i], pkbuf.at[s],
                                          psem[s]).wait()

                def gather_start(i, s, b):
                    for kk in range(CHUNK // 16):
                        sl = pl.ds(kk * 16, 16)
                        w = pkbuf[s, sl]
                        sbuf[b, sl] = lax.bitwise_and(w, 0xFFFF)
                        dbuf[b, sl] = lax.shift_right_arithmetic(w, 16)
                    idx = plsc.Indices(sbuf.at[b], ignored_value=0xFFFF)
                    pltpu.async_copy(hn_ref.at[g].at[idx], grows(b), gsem[b])

                def gather_wait(i, b):
                    idx = plsc.Indices(sbuf.at[b], ignored_value=0xFFFF)
                    pltpu.make_async_copy(hn_ref.at[g].at[idx], grows(b),
                                          gsem[b]).wait()

                def scatter_start(i, b):
                    dst = acc.at[plsc.Indices(dbuf.at[b], ignored_value=-1)]
                    pltpu.async_copy(grows(b), dst, ssem[b], add=True)

                def scatter_wait(i, b):
                    dst = acc.at[plsc.Indices(dbuf.at[b], ignored_value=-1)]
                    pltpu.make_async_copy(grows(b), dst, ssem[b]).wait()

                Q = CPT // PB

                def round_(j, first, last):
                    base = j * PB
                    for h in range(2):
                        for k in range(h * NBUF, (h + 1) * NBUF):
                            i = base + k
                            b = k % NBUF
                            if h == 0:
                                if not first:
                                    scatter_wait(base - PB + NBUF + k, b)
                            else:
                                scatter_wait(i - NBUF, b)
                            pk_wait(i, k)
                            gather_start(i, k, b)
                            if not last:
                                pk_start(i + PB, k)
                        for k in range(h * NBUF, (h + 1) * NBUF):
                            i = base + k
                            b = k % NBUF
                            gather_wait(i, b)
                            scatter_start(i, b)

                for s in range(PB):
                    pk_start(s, s)
                round_(0, True, False)

                def estep(j, _):
                    round_(j, False, False)
                    return 0

                lax.fori_loop(1, Q - 1, estep, 0)
                round_(Q - 1, False, True)
                for b in range(NBUF):
                    scatter_wait((Q - 1) * PB + NBUF + b, b)
                plsc.subcore_barrier()

                def wa(i, b, wait=False):
                    d = pltpu.make_async_copy(acc.at[arange_(i)], wrows(b),
                                              gsem[b])
                    d.wait() if wait else d.start()

                def wb(i, b, wait=False):
                    d = pltpu.make_async_copy(wrows(b),
                                              out_ref.at[g, rrange(i)],
                                              ssem[b])
                    d.wait() if wait else d.start()

                _two_hop(RPR // WCH, NBUF,
                         lambda i, b: wa(i, b), lambda i, b: wa(i, b, True),
                         lambda i, b: wb(i, b), lambda i, b: wb(i, b, True))
                plsc.subcore_barrier()

    return k(hn, packed)


def _prep_edges(edge_index):
    e = edge_index.shape[1]
    src = jnp.pad(edge_index[0], (0, E_PAD - e), constant_values=0)
    dst = jnp.pad(edge_index[1], (0, E_PAD - e), constant_values=-1)
    rid = jnp.arange(NRANGE, dtype=jnp.int32)[:, None]
    in_rng = (dst[None, :] >= rid * RNG) & (dst[None, :] < (rid + 1) * RNG)
    word = jnp.where(in_rng,
                     ((dst[None, :] - rid * RNG) << 16) | src[None, :],
                     jnp.int32(-1))
    return (dst.reshape(16, CPT, CHUNK),
            word.reshape(NRANGE, 16, CPT, CHUNK))


def _tower(x, packed, deg, W1, b1, W2, b2, batch):
    hn = _tc_first(x, W1, deg)
    A = _sc_agg(hn, packed)
    hn = _tc_mid(A, W2[0], b1.reshape(1, -1), deg)
    A = _sc_agg(hn, packed)
    hn = _tc_mid(A, W2[1], b2[0].reshape(1, -1), deg)
    A = _sc_agg(hn, packed)
    return _tc_pool(A, b2[1].reshape(1, -1), deg, batch)


def kernel(x_s, edge_index_s, x_p, edge_index_p, x_s_batch, x_p_batch,
           dose, time, drop_pert,
           W_c1, b_c1, W_c2, b_c2, W_g1, b_g1, W_g2, b_g2,
           W_lx, b_lx, W_le, b_le, W_ld, b_ld, W_lt, b_lt,
           W_l1, b_l1, W_l2, b_l2):
    n_s = x_s.shape[0]
    n_p = x_p.shape[0]
    xs = jnp.pad(x_s, ((0, N_PAD - n_s), (0, 0)))
    xp = jnp.pad(x_p, ((0, N_PAD - n_p), (0, 6)))
    Wg1 = jnp.pad(W_g1, ((0, 6), (0, 0)))
    bat_s = jnp.pad(x_s_batch, (0, N_PAD - n_s),
                    constant_values=NB).reshape(N_PAD, 1)
    bat_p = jnp.pad(x_p_batch, (0, N_PAD - n_p),
                    constant_values=NB).reshape(N_PAD, 1)
    dst_s2, packed_s = _prep_edges(edge_index_s)
    dst_p2, packed_p = _prep_edges(edge_index_p)

    deg_s, deg_p = _sc_degree(dst_s2, dst_p2)
    deg_s = deg_s.reshape(N_PAD, 1)
    deg_p = deg_p.reshape(N_PAD, 1)

    sums_s, cnt_s = _tower(xs, packed_s, deg_s, W_c1, b_c1, W_c2, b_c2,
                           bat_s)
    sums_p, cnt_p = _tower(xp, packed_p, deg_p, Wg1, b_g1, W_g2, b_g2,
                           bat_p)

    OUT = W_l2.shape[1]
    OUTP = ((OUT + 127) // 128) * 128
    W_l2p = jnp.pad(W_l2, ((0, 0), (0, OUTP - OUT)))
    b_l2p = jnp.pad(b_l2, (0, OUTP - OUT)).reshape(1, OUTP)

    D = W_lx.shape[1]
    out = _tc_head(sums_s, cnt_s, sums_p, cnt_p, dose, time,
                   W_lx, b_lx.reshape(1, -1), W_le, b_le.reshape(1, -1),
                   W_ld, b_ld.reshape(1, 1), W_lt, b_lt.reshape(1, 1),
                   W_l1[:D], W_l1[D:2 * D], W_l1[2 * D:2 * D + 1],
                   W_l1[2 * D + 1:2 * D + 2], b_l1.reshape(1, -1),
                   W_l2p, b_l2p)
    return out[:, :OUT]


# speedup vs baseline: 6.2634x; 1.0238x over previous
import functools

import jax
import jax.numpy as jnp
from jax import lax
from jax.experimental import pallas as pl
from jax.experimental.pallas import tpu as pltpu
from jax.experimental.pallas import tpu_sc as plsc

def _elu(x):
    return jnp.where(x > 0, x, jnp.exp(jnp.minimum(x, 0.0)) - 1.0)


N_PAD = 51200
RPT = N_PAD // 16
BM = 512
CHUNK = 128
CPT = 392
E_PAD = 16 * CPT * CHUNK
NB = 64
NRANGE = 8
RNG = N_PAD // NRANGE
RPR = RNG // 16
WCH = 80
NBUF = 4


def _store_groups(o_ref, h, ng):
    for g in range(ng):
        o_ref[g] = h[:, g * 128:(g + 1) * 128]


def _load_groups(a_ref, ng):
    if ng == 1:
        return a_ref[0]
    return jnp.concatenate([a_ref[g] for g in range(ng)], axis=1)


def _tc_first(x, W, deg):
    F = W.shape[1]
    NG = F // 128

    def body(x_ref, w_ref, deg_ref, o_ref):
        dinv = lax.rsqrt(deg_ref[...] + 1.0)
        h = jnp.dot(x_ref[...], w_ref[...], preferred_element_type=jnp.float32)
        _store_groups(o_ref, h * dinv, NG)

    return pl.pallas_call(
        body,
        grid=(N_PAD // BM,),
        in_specs=[
            pl.BlockSpec((BM, x.shape[1]), lambda i: (i, 0)),
            pl.BlockSpec(W.shape, lambda i: (0, 0)),
            pl.BlockSpec((BM, 1), lambda i: (i, 0)),
        ],
        out_specs=pl.BlockSpec((NG, BM, 128), lambda i: (0, i, 0)),
        out_shape=jax.ShapeDtypeStruct((NG, N_PAD, 128), jnp.float32),
    )(x, W, deg)


def _tc_mid(A, W, b, deg):
    FI = W.shape[0]
    F = W.shape[1]
    NGI = FI // 128
    NG = F // 128

    def body(a_ref, w_ref, b_ref, deg_ref, o_ref):
        dinv = lax.rsqrt(deg_ref[...] + 1.0)
        act = _elu(_load_groups(a_ref, NGI) * dinv + b_ref[...])
        h = jnp.dot(act, w_ref[...], preferred_element_type=jnp.float32)
        _store_groups(o_ref, h * dinv, NG)

    return pl.pallas_call(
        body,
        grid=(N_PAD // BM,),
        in_specs=[
            pl.BlockSpec((NGI, BM, 128), lambda i: (0, i, 0)),
            pl.BlockSpec(W.shape, lambda i: (0, 0)),
            pl.BlockSpec((1, F), lambda i: (0, 0)),
            pl.BlockSpec((BM, 1), lambda i: (i, 0)),
        ],
        out_specs=pl.BlockSpec((NG, BM, 128), lambda i: (0, i, 0)),
        out_shape=jax.ShapeDtypeStruct((NG, N_PAD, 128), jnp.float32),
    )(A, W, b, deg)


def _tc_pool(A, b, deg, batch):
    NGI = A.shape[0]
    F = NGI * 128

    def body(a_ref, b_ref, deg_ref, bat_ref, sums_ref, cnt_ref):
        i = pl.program_id(0)

        @pl.when(i == 0)
        def _():
            sums_ref[...] = jnp.zeros_like(sums_ref)
            cnt_ref[...] = jnp.zeros_like(cnt_ref)

        dinv = lax.rsqrt(deg_ref[...] + 1.0)
        act = _elu(_load_groups(a_ref, NGI) * dinv + b_ref[...])
        ids = lax.broadcasted_iota(jnp.int32, (BM, NB), 1)
        oh = (bat_ref[...] == ids).astype(jnp.float32)
        dn = (((0,), (0,)), ((), ()))
        sums_ref[...] += lax.dot_general(oh, act, dn,
                                         preferred_element_type=jnp.float32)
        cnt_ref[...] += lax.dot_general(oh, jnp.ones((BM, 1), jnp.float32), dn,
                                        preferred_element_type=jnp.float32)

    return pl.pallas_call(
        body,
        grid=(N_PAD // BM,),
        in_specs=[
            pl.BlockSpec((NGI, BM, 128), lambda i: (0, i, 0)),
            pl.BlockSpec((1, F), lambda i: (0, 0)),
            pl.BlockSpec((BM, 1), lambda i: (i, 0)),
            pl.BlockSpec((BM, 1), lambda i: (i, 0)),
        ],
        out_specs=[
            pl.BlockSpec((NB, F), lambda i: (0, 0)),
            pl.BlockSpec((NB, 1), lambda i: (0, 0)),
        ],
        out_shape=[
            jax.ShapeDtypeStruct((NB, F), jnp.float32),
            jax.ShapeDtypeStruct((NB, 1), jnp.float32),
        ],
    )(A, b, deg, batch)


def _tc_head(sums_s, cnt_s, sums_p, cnt_p, dose, time,
             W_lx, b_lx, W_le, b_le, W_ld, b_ld, W_lt, b_lt,
             W1x, W1e, w1d, w1t, b_l1, W_l2p, b_l2p):
    OUTP = W_l2p.shape[1]

    def body(ss_ref, cs_ref, sp_ref, cp_ref, do_ref, ti_ref,
             wlx_ref, blx_ref, wle_ref, ble_ref, wld_ref, bld_ref,
             wlt_ref, blt_ref, w1x_ref, w1e_ref, w1d_ref, w1t_ref,
             bl1_ref, wl2_ref, bl2_ref, o_ref):
        xs = ss_ref[...] / jnp.maximum(cs_ref[...], 1.0)
        xp = sp_ref[...] / jnp.maximum(cp_ref[...], 1.0)
        x = jnp.dot(xs, wlx_ref[...], preferred_element_type=jnp.float32) \
            + blx_ref[...]
        e = jnp.dot(xp, wle_ref[...], preferred_element_type=jnp.float32) \
            + ble_ref[...]
        d = do_ref[...] * wld_ref[...] + bld_ref[...]
        t = ti_ref[...] * wlt_ref[...] + blt_ref[...]
        h = (jnp.dot(x, w1x_ref[...], preferred_element_type=jnp.float32)
             + jnp.dot(e, w1e_ref[...], preferred_element_type=jnp.float32)
             + jnp.dot(d, w1d_ref[...], preferred_element_type=jnp.float32)
             + jnp.dot(t, w1t_ref[...], preferred_element_type=jnp.float32)
             + bl1_ref[...])
        h = _elu(h)
        o_ref[...] = jnp.dot(h, wl2_ref[...],
                             preferred_element_type=jnp.float32) + bl2_ref[...]

    return pl.pallas_call(
        body,
        out_shape=jax.ShapeDtypeStruct((NB, OUTP), jnp.float32),
    )(sums_s, cnt_s, sums_p, cnt_p, dose, time,
      W_lx, b_lx, W_le, b_le, W_ld, b_ld, W_lt, b_lt,
      W1x, W1e, w1d, w1t, b_l1, W_l2p, b_l2p)


_MESH = dict(core_axis_name="c", subcore_axis_name="s")


def _sc_degree(dst_s2, dst_p2):
    mesh = plsc.VectorSubcoreMesh(**_MESH)

    @functools.partial(
        pl.kernel,
        out_type=(jax.ShapeDtypeStruct((N_PAD,), jnp.float32),
                  jax.ShapeDtypeStruct((N_PAD,), jnp.float32)),
        mesh=mesh,
        scratch_types=[
            pltpu.VMEM((CPT, CHUNK), jnp.int32),
            pltpu.VMEM((RPT,), jnp.float32),
            pltpu.VMEM((CHUNK,), jnp.float32),
            pltpu.VMEM_SHARED((N_PAD,), jnp.float32),
            pltpu.SemaphoreType.DMA,
        ],
    )
    def k(ds_ref, dp_ref, degs_ref, degp_ref, idx_v, zbuf, obuf, acc, sem):
        c = lax.axis_index("c")
        t = lax.axis_index("s")

        def fz(i, _):
            zbuf[pl.ds(i * 16, 16)] = jnp.zeros((16,), jnp.float32)
            return 0

        lax.fori_loop(0, RPT // 16, fz, 0)
        for i in range(CHUNK // 16):
            obuf[pl.ds(i * 16, 16)] = jnp.ones((16,), jnp.float32)
        pltpu.sync_copy(zbuf, acc.at[pl.ds(t * RPT, RPT)])

        for core in range(2):
            @pl.when(c == core)
            def _(core=core):
                din = ds_ref if core == 0 else dp_ref
                dout = degs_ref if core == 0 else degp_ref
                pltpu.sync_copy(din.at[t], idx_v)
                plsc.subcore_barrier()

                def fs(o, _):
                    descs = []
                    for b in range(8):
                        dst = acc.at[plsc.Indices(idx_v.at[o * 8 + b],
                                                  ignored_value=-1)]
                        descs.append(pltpu.async_copy(obuf, dst, sem,
                                                      add=True))
                    for d in descs:
                        d.wait()
                    return 0

                lax.fori_loop(0, CPT // 8, fs, 0)
                for i in range((CPT // 8) * 8, CPT):
                    dst = acc.at[plsc.Indices(idx_v.at[i], ignored_value=-1)]
                    pltpu.async_copy(obuf, dst, sem, add=True).wait()
                plsc.subcore_barrier()
                pltpu.sync_copy(acc.at[pl.ds(t * RPT, RPT)], zbuf)
                pltpu.sync_copy(zbuf, dout.at[pl.ds(t * RPT, RPT)])

    return k(dst_s2, dst_p2)


def _two_hop(n_chunks, nbuf, a_start, a_wait, b_start, b_wait):
    q = n_chunks // nbuf
    r = n_chunks - q * nbuf
    if q > 0:
        for b in range(nbuf):
            a_start(b, b)
        if q > 1:
            def step(j, _):
                base = j * nbuf
                for b in range(nbuf):
                    b_wait(base - nbuf + b, b)
                    a_start(base + b, b)
                for b in range(nbuf):
                    a_wait(base + b, b)
                    b_start(base + b, b)
                return 0

            for b in range(nbuf):
                a_wait(b, b)
                b_start(b, b)
            lax.fori_loop(1, q, step, 0)
        else:
            for b in range(nbuf):
                a_wait(b, b)
                b_start(b, b)
        for b in range(nbuf):
            b_wait((q - 1) * nbuf + b, b)
    for i in range(r):
        c = q * nbuf + i
        a_start(c, i)
        a_wait(c, i)
        b_start(c, i)
        b_wait(c, i)


def _three_hop(n_chunks, nbuf, a_start, a_wait, b_start, b_wait,
               c_start, c_wait):
    q = n_chunks // nbuf
    assert q >= 2 and n_chunks == q * nbuf

    def round_(j, first):
        base = j * nbuf
        for b in range(nbuf):
            if not first:
                c_wait(base - nbuf + b, b)
            a_start(base + b, b)
        for b in range(nbuf):
            a_wait(base + b, b)
            b_start(base + b, b)
        for b in range(nbuf):
            b_wait(base + b, b)
            c_start(base + b, b)

    round_(0, True)

    def step(j, _):
        round_(j, False)
        return 0

    lax.fori_loop(1, q, step, 0)
    for b in range(nbuf):
        c_wait((q - 1) * nbuf + b, b)


def _sc_agg(hn, packed):
    NG = hn.shape[0]
    mesh = plsc.VectorSubcoreMesh(**_MESH)
    PB = 2 * NBUF

    @functools.partial(
        pl.kernel,
        out_type=jax.ShapeDtypeStruct((NG, N_PAD, 128), jnp.float32),
        mesh=mesh,
        scratch_types=[
            pltpu.VMEM((PB, CHUNK), jnp.int32),
            pltpu.VMEM((NBUF, CHUNK), jnp.int32),
            pltpu.VMEM((NBUF, CHUNK), jnp.int32),
            pltpu.VMEM((NBUF, CHUNK, 128), jnp.float32),
            pltpu.VMEM_SHARED((RNG, 128), jnp.float32),
        ] + [pltpu.SemaphoreType.DMA] * (PB + 2 * NBUF),
    )
    def k(hn_ref, pk_ref, out_ref, pkbuf, sbuf, dbuf, rows, acc, *sems):
        psem = sems[:PB]
        gsem = sems[PB:PB + NBUF]
        ssem = sems[PB + NBUF:]
        c = lax.axis_index("c")
        t = lax.axis_index("s")

        for ri in range(NRANGE // 2):
            r = c * (NRANGE // 2) + ri

            for g in range(NG):
                def rrange(i):
                    return pl.ds(r * RNG + t * RPR + i * WCH, WCH)

                def arange_(i):
                    return pl.ds(t * RPR + i * WCH, WCH)

                def wrows(b):
                    return rows.at[b].at[pl.ds(0, WCH)]

                def ia(i, b, wait=False):
                    d = pltpu.make_async_copy(hn_ref.at[g, rrange(i)],
                                              wrows(b), gsem[b])
                    d.wait() if wait else d.start()

                def ib(i, b, wait=False):
                    d = pltpu.make_async_copy(wrows(b), acc.at[arange_(i)],
                                              ssem[b])
                    d.wait() if wait else d.start()

                _two_hop(RPR // WCH, NBUF,
                         lambda i, b: ia(i, b), lambda i, b: ia(i, b, True),
                         lambda i, b: ib(i, b), lambda i, b: ib(i, b, True))
                plsc.subcore_barrier()

                def grows(b):
                    return rows.at[b]

                def pk_start(i, s):
                    pltpu.async_copy(pk_ref.at[r, t, i], pkbuf.at[s],
                                     psem[s])

                def pk_wait(i, s):
                    pltpu.make_async_copy(pk_ref.at[r, t, i], pkbuf.at[s],
                                          psem[s]).wait()

                def gather_start(i, s, b):
                    for kk in range(CHUNK // 16):
                        sl = pl.ds(kk * 16, 16)
                        w = pkbuf[s, sl]
                        sbuf[b, sl] = lax.bitwise_and(w, 0xFFFF)
                        dbuf[b, sl] = lax.shift_right_arithmetic(w, 16)
                    idx = plsc.Indices(sbuf.at[b], ignored_value=0xFFFF)
                    pltpu.async_copy(hn_ref.at[g].at[idx], grows(b), gsem[b])

                def gather_wait(i, b):
                    idx = plsc.Indices(sbuf.at[b], ignored_value=0xFFFF)
                    pltpu.make_async_copy(hn_ref.at[g].at[idx], grows(b),
                                          gsem[b]).wait()

                def scatter_start(i, b):
                    dst = acc.at[plsc.Indices(dbuf.at[b], ignored_value=-1)]
                    pltpu.async_copy(grows(b), dst, ssem[b], add=True)

                def scatter_wait(i, b):
                    dst = acc.at[plsc.Indices(dbuf.at[b], ignored_value=-1)]
                    pltpu.make_async_copy(grows(b), dst, ssem[b]).wait()

                Q = CPT // PB

                def round_(j, first, last):
                    base = j * PB
                    for h in range(2):
                        for k in range(h * NBUF, (h + 1) * NBUF):
                            i = base + k
                            b = k % NBUF
                            if h == 0:
                                if not first:
                                    scatter_wait(base - PB + NBUF + k, b)
                            else:
                                scatter_wait(i - NBUF, b)
                            pk_wait(i, k)
                            gather_start(i, k, b)
                            if not last:
                                pk_start(i + PB, k)
                        for k in range(h * NBUF, (h + 1) * NBUF):
                            i = base + k
                            b = k % NBUF
                            gather_wait(i, b)
                            scatter_start(i, b)

                for s in range(PB):
                    pk_start(s, s)
                round_(0, True, False)

                def estep(j, _):
                    round_(j, False, False)
                    return 0

                lax.fori_loop(1, Q - 1, estep, 0)
                round_(Q - 1, False, True)
                for b in range(NBUF):
                    scatter_wait((Q - 1) * PB + NBUF + b, b)
                plsc.subcore_barrier()

                def wa(i, b, wait=False):
                    d = pltpu.make_async_copy(acc.at[arange_(i)], wrows(b),
                                              gsem[b])
                    d.wait() if wait else d.start()

                def wb(i, b, wait=False):
                    d = pltpu.make_async_copy(wrows(b),
                                              out_ref.at[g, rrange(i)],
                                              ssem[b])
                    d.wait() if wait else d.start()

                _two_hop(RPR // WCH, NBUF,
                         lambda i, b: wa(i, b), lambda i, b: wa(i, b, True),
                         lambda i, b: wb(i, b), lambda i, b: wb(i, b, True))
                plsc.subcore_barrier()

    return k(hn, packed)


def _prep_edges(edge_index):
    e = edge_index.shape[1]
    src = jnp.pad(edge_index[0], (0, E_PAD - e), constant_values=0)
    dst = jnp.pad(edge_index[1], (0, E_PAD - e), constant_values=-1)
    rid = jnp.arange(NRANGE, dtype=jnp.int32)[:, None]
    in_rng = (dst[None, :] >= rid * RNG) & (dst[None, :] < (rid + 1) * RNG)
    word = jnp.where(in_rng,
                     ((dst[None, :] - rid * RNG) << 16) | src[None, :],
                     jnp.int32(-1))
    return (dst.reshape(16, CPT, CHUNK),
            word.reshape(NRANGE, 16, CPT, CHUNK))


def _tower(x, packed, deg, W1, b1, W2, b2, batch):
    hn = _tc_first(x, W1, deg)
    A = _sc_agg(hn, packed)
    hn = _tc_mid(A, W2[0], b1.reshape(1, -1), deg)
    A = _sc_agg(hn, packed)
    hn = _tc_mid(A, W2[1], b2[0].reshape(1, -1), deg)
    A = _sc_agg(hn, packed)
    return _tc_pool(A, b2[1].reshape(1, -1), deg, batch)


def kernel(x_s, edge_index_s, x_p, edge_index_p, x_s_batch, x_p_batch,
           dose, time, drop_pert,
           W_c1, b_c1, W_c2, b_c2, W_g1, b_g1, W_g2, b_g2,
           W_lx, b_lx, W_le, b_le, W_ld, b_ld, W_lt, b_lt,
           W_l1, b_l1, W_l2, b_l2):
    n_s = x_s.shape[0]
    n_p = x_p.shape[0]
    xs = jnp.pad(x_s, ((0, N_PAD - n_s), (0, 0)))
    xp = jnp.pad(x_p, ((0, N_PAD - n_p), (0, 6)))
    Wg1 = jnp.pad(W_g1, ((0, 6), (0, 0)))
    bat_s = jnp.pad(x_s_batch, (0, N_PAD - n_s),
                    constant_values=NB).reshape(N_PAD, 1)
    bat_p = jnp.pad(x_p_batch, (0, N_PAD - n_p),
                    constant_values=NB).reshape(N_PAD, 1)
    dst_s2, packed_s = _prep_edges(edge_index_s)
    dst_p2, packed_p = _prep_edges(edge_index_p)

    deg_s, deg_p = _sc_degree(dst_s2, dst_p2)
    deg_s = deg_s.reshape(N_PAD, 1)
    deg_p = deg_p.reshape(N_PAD, 1)

    sums_s, cnt_s = _tower(xs, packed_s, deg_s, W_c1, b_c1, W_c2, b_c2,
                           bat_s)
    sums_p, cnt_p = _tower(xp, packed_p, deg_p, Wg1, b_g1, W_g2, b_g2,
                           bat_p)

    OUT = W_l2.shape[1]
    OUTP = ((OUT + 127) // 128) * 128
    W_l2p = jnp.pad(W_l2, ((0, 0), (0, OUTP - OUT)))
    b_l2p = jnp.pad(b_l2, (0, OUTP - OUT)).reshape(1, OUTP)

    D = W_lx.shape[1]
    out = _tc_head(sums_s, cnt_s, sums_p, cnt_p, dose, time,
                   W_lx, b_lx.reshape(1, -1), W_le, b_le.reshape(1, -1),
                   W_ld, b_ld.reshape(1, 1), W_lt, b_lt.reshape(1, 1),
                   W_l1[:D], W_l1[D:2 * D], W_l1[2 * D:2 * D + 1],
                   W_l1[2 * D + 1:2 * D + 2], b_l1.reshape(1, -1),
                   W_l2p, b_l2p)
    return out[:, :OUT]
